# baseline (speedup 1.0000x reference)
import functools

import jax
import jax.numpy as jnp
from jax import lax
from jax.experimental import pallas as pl
from jax.experimental.pallas import tpu as pltpu
from jax.experimental.pallas import tpu_sc as plsc

HIDDEN = 32
OUT = 16
CB = 1152
V = 1000000
NB = 999936 // CB
VMAIN = NB * CB
NTAIL = V - VMAIN
B = 16384
SENT = 63 * 2048


def _sc_stream_gather(embt, owner, payload):
    info = plsc.get_sparse_core_info()
    nc, ns = info.num_cores, info.num_subcores
    nw = nc * ns
    mesh = plsc.VectorSubcoreMesh(core_axis_name="c", subcore_axis_name="s")
    n_stream = B // 2048

    @functools.partial(
        pl.kernel,
        mesh=mesh,
        out_type=jax.ShapeDtypeStruct((B + 16, 128), jnp.float32),
        scratch_types=[
            pltpu.VMEM((HIDDEN, CB), jnp.float32),
            pltpu.VMEM((HIDDEN, CB), jnp.float32),
            pltpu.VMEM((2048,), jnp.int32),
            pltpu.VMEM((2048,), jnp.int32),
            pltpu.VMEM((B + 48,), jnp.int32),
            pltpu.VMEM((B + 48,), jnp.int32),
            pltpu.VMEM((16, 128), jnp.float32),
            pltpu.VMEM((16, 128), jnp.float32),
            pltpu.SemaphoreType.DMA,
            pltpu.SemaphoreType.DMA,
            pltpu.SemaphoreType.DMA,
            pltpu.SemaphoreType.DMA,
            pltpu.SemaphoreType.DMA,
        ],
        compiler_params=pltpu.CompilerParams(needs_layout_passes=False),
    )
    def k(embt_hbm, own_hbm, pay_hbm, h4_hbm,
          buf0, buf1, ochunk, pchunk, jl, plist, rows0, rows1,
          rs0, rs1, ps, ss0, ss1):
        wid = lax.axis_index("s") * nc + lax.axis_index("c")
        n_my = (NB - wid + nw - 1) >> 5
        bufs = [buf0, buf1]
        rsems = [rs0, rs1]
        rowbufs = [rows0, rows1]
        ssems = [ss0, ss1]
        iota16 = lax.iota(jnp.int32, 16)

        def p1_outer(s, cur):
            pltpu.sync_copy(own_hbm.at[pl.ds(s * 2048, 2048)], ochunk)
            pltpu.sync_copy(pay_hbm.at[pl.ds(s * 2048, 2048)], pchunk)

            def p1_inner(t, cur):
                sl = pl.ds(t * 16, 16)
                o = ochunk[sl]
                p = pchunk[sl]
                m = o == wid
                jv = (s * 2048 + t * 16) + iota16
                mi = jnp.where(m, 1, 0)
                pf = plsc.cumsum(mi)
                pos = jnp.where(m, cur + pf - 1, B + 40)
                plsc.store_scatter(jl, [pos], jv)
                plsc.store_scatter(plist, [pos], p)
                return cur + jnp.sum(mi)

            return lax.fori_loop(0, 128, p1_inner, cur)

        cur = lax.fori_loop(0, n_stream, p1_outer, 0)
        sent = jnp.full((16,), SENT, jnp.int32)
        plist[pl.ds(cur, 16)] = sent
        plist[pl.ds(cur + 16, 16)] = sent
        npairs = (cur >> 5) + 1

        def read_block(g, par):
            c0 = pl.multiple_of(g * CB, 128)
            pltpu.async_copy(embt_hbm.at[:, pl.ds(c0, CB)], bufs[par], rsems[par])

        def wait_block(g, par):
            c0 = pl.multiple_of(g * CB, 128)
            pltpu.make_async_copy(
                embt_hbm.at[:, pl.ds(c0, CB)], bufs[par], rsems[par]
            ).wait()

        def drain_scatter(par):
            pltpu.make_async_copy(
                rowbufs[par], h4_hbm.at[pl.ds(0, 16)], ssems[par]
            ).wait()

        def process_block(i_blk, bpar, has):
            buf = bufs[bpar]

            def pair_body(kp, has):
                def half(par, has_p):
                    sl = pl.ds(kp * 32 + par * 16, 16)
                    jv = jl[sl]
                    pv = plist[sl]
                    sel = (pv >> 11) == i_blk
                    anysel = jnp.any(sel)

                    @pl.when(anysel)
                    def _do():
                        @pl.when(has_p)
                        def _dr():
                            drain_scatter(par)

                        col = pv & 2047
                        cvec = jnp.zeros((16,), jnp.int32)
                        for c in range(HIDDEN):
                            vals = plsc.load_gather(buf, [cvec + c, col])
                            plsc.store_scatter(
                                rowbufs[par], [iota16, cvec + c], vals
                            )
                        jsafe = jnp.where(sel, jv, B + iota16)
                        pltpu.async_copy(
                            rowbufs[par], h4_hbm.at[jsafe], ssems[par]
                        )

                    return jnp.where(anysel, True, has_p)

                h0 = half(0, has[0])
                h1 = half(1, has[1])
                return (h0, h1)

            return lax.fori_loop(0, npairs, pair_body, has)

        def outer(i2, has):
            i0 = 2 * i2
            g0 = wid + i0 * nw
            g1 = g0 + nw

            @pl.when(i2 == 0)
            def _prime():
                read_block(g0, 0)

            @pl.when(i0 + 1 < n_my)
            def _pref1():
                read_block(g1, 1)

            wait_block(g0, 0)
            has = process_block(i0, 0, has)

            @pl.when(i0 + 2 < n_my)
            def _pref2():
                read_block(g0 + 2 * nw, 0)

            def do_second(has):
                wait_block(g1, 1)
                return process_block(i0 + 1, 1, has)

            has = lax.cond(i0 + 1 < n_my, do_second, lambda h: h, has)
            return has

        n_out = (n_my + 1) >> 1
        has = lax.fori_loop(0, n_out, outer, (False, False))

        @pl.when(has[0])
        def _f0():
            drain_scatter(0)

        @pl.when(has[1])
        def _f1():
            drain_scatter(1)

    return k(embt, owner, payload)


def _mlp_body(h4_ref, idx_ref, tail_ref, w_ref, b_ref, o_ref):
    h = h4_ref[:B, :HIDDEN]
    idx = idx_ref[...]
    tail_sel = idx - VMAIN
    onehot = jnp.where(
        lax.broadcasted_iota(jnp.int32, (B, NTAIL), 1) == tail_sel, 1.0, 0.0
    )
    htail = lax.dot_general(
        onehot, tail_ref[...], (((1,), (0,)), ((), ())),
        preferred_element_type=jnp.float32,
    )
    hs = jnp.where(idx >= VMAIN, htail, h)
    hr = jnp.maximum(hs, 0.0)
    o_ref[...] = (
        lax.dot_general(
            hr, w_ref[...], (((1,), (1,)), ((), ())),
            preferred_element_type=jnp.float32,
        )
        + b_ref[...]
    )


def kernel(x, emb, W2, b2):
    b = x.shape[0]
    idx = x.reshape(b).astype(jnp.int32)
    idx_c = jnp.minimum(idx, VMAIN - 1)
    g = idx_c // CB
    col = idx_c - g * CB
    owner = g & 31
    payload = (g >> 5) * 2048 + col

    h4 = _sc_stream_gather(emb.T, owner, payload)

    tail = emb[VMAIN:]
    y = pl.pallas_call(
        _mlp_body,
        out_shape=jax.ShapeDtypeStruct((b, OUT), jnp.float32),
    )(h4, idx.reshape(b, 1), tail, W2, b2.reshape(1, OUT))
    return y


# speedup vs baseline: 4.6207x; 4.6207x over previous
import functools

import jax
import jax.numpy as jnp
from jax import lax
from jax.experimental import pallas as pl
from jax.experimental.pallas import tpu as pltpu
from jax.experimental.pallas import tpu_sc as plsc

HIDDEN = 32
OUT = 16
CB = 768
BLK = HIDDEN * CB
V = 1000000
NB = 999936 // CB
VMAIN = NB * CB
NTAIL = V - VMAIN
FLAT_N = NB * BLK


def _sc_detile(embt):
    info = plsc.get_sparse_core_info()
    nc, ns = info.num_cores, info.num_subcores
    nw = nc * ns
    mesh = plsc.VectorSubcoreMesh(core_axis_name="c", subcore_axis_name="s")

    @functools.partial(
        pl.kernel,
        mesh=mesh,
        out_type=jax.ShapeDtypeStruct((FLAT_N,), jnp.float32),
        scratch_types=[
            pltpu.VMEM((HIDDEN, CB), jnp.float32),
            pltpu.VMEM((HIDDEN, CB), jnp.float32),
            pltpu.VMEM((HIDDEN, CB), jnp.float32),
            pltpu.VMEM((HIDDEN, CB), jnp.float32),
            pltpu.SemaphoreType.DMA,
            pltpu.SemaphoreType.DMA,
            pltpu.SemaphoreType.DMA,
            pltpu.SemaphoreType.DMA,
            pltpu.SemaphoreType.DMA,
            pltpu.SemaphoreType.DMA,
            pltpu.SemaphoreType.DMA,
            pltpu.SemaphoreType.DMA,
        ],
    )
    def detile_kernel(embt_hbm, flat_hbm, b0, b1, b2, b3,
                      r0s, r1s, r2s, r3s, w0s, w1s, w2s, w3s):
        wid = lax.axis_index("s") * nc + lax.axis_index("c")
        n_my = (NB - wid + nw - 1) // nw
        bufs = [b0, b1, b2, b3]
        rsems = [r0s, r1s, r2s, r3s]
        wsems = [w0s, w1s, w2s, w3s]

        def read_block(g, buf, rsem):
            c0 = pl.multiple_of(g * CB, 128)
            return pltpu.async_copy(embt_hbm.at[:, pl.ds(c0, CB)], buf, rsem)

        def write_descs(g, buf, wsem):
            o0 = pl.multiple_of(g * BLK, 8)
            return [
                pltpu.make_async_copy(
                    buf.at[c], flat_hbm.at[pl.ds(o0 + c * CB, CB)], wsem
                )
                for c in range(HIDDEN)
            ]

        def body(i, _):
            for k in range(4):
                g = wid + (4 * i + k) * nw

                @pl.when(i > 0)
                def _drain():
                    for cp in write_descs(g, bufs[k], wsems[k]):
                        cp.wait()

                read_block(g, bufs[k], rsems[k])
            for k in range(4):
                g = wid + (4 * i + k) * nw
                pltpu.make_async_copy(
                    embt_hbm.at[:, pl.ds(pl.multiple_of(g * CB, 128), CB)],
                    bufs[k],
                    rsems[k],
                ).wait()
                for cp in write_descs(g, bufs[k], wsems[k]):
                    cp.start()
            return _

        n4 = n_my // 4
        lax.fori_loop(0, n4, body, None)
        for k in range(4):
            @pl.when(n4 > 0)
            def _final_drain(k=k):
                g = wid + (4 * (n4 - 1) + k) * nw
                for cp in write_descs(g, bufs[k], wsems[k]):
                    cp.wait()

        @pl.when((n_my % 4) == 1)
        def _tail():
            g = wid + (n_my - 1) * nw
            read_block(g, bufs[0], rsems[0]).wait()
            for cp in write_descs(g, bufs[0], wsems[0]):
                cp.start()
            for cp in write_descs(g, bufs[0], wsems[0]):
                cp.wait()

    return detile_kernel(embt)


def _sc_gather_t(flat, pos0):
    info = plsc.get_sparse_core_info()
    nc, ns = info.num_cores, info.num_subcores
    nw = nc * ns
    b = pos0.shape[0]
    assert b % (8 * nw) == 0
    b_per_w = b // nw
    n_vec = b_per_w // 16
    mesh = plsc.VectorSubcoreMesh(core_axis_name="c", subcore_axis_name="s")

    @functools.partial(
        pl.kernel,
        mesh=mesh,
        out_type=jax.ShapeDtypeStruct((HIDDEN, b), jnp.float32),
        scratch_types=[
            pltpu.VMEM((b_per_w,), jnp.int32),
            pltpu.VMEM((HIDDEN, b_per_w), jnp.int32),
            pltpu.VMEM((HIDDEN, b_per_w), jnp.float32),
            pltpu.SemaphoreType.DMA,
        ],
        compiler_params=pltpu.CompilerParams(use_tc_tiling_on_sc=False),
    )
    def gather_kernel(flat_hbm, pos_hbm, out_hbm, pos_v, idxc_v, rows_v, sem):
        wid = lax.axis_index("s") * nc + lax.axis_index("c")
        base = wid * b_per_w
        pltpu.sync_copy(pos_hbm.at[pl.ds(base, b_per_w)], pos_v)

        def build(t, _):
            sl = pl.ds(t * 16, 16)
            acc = pos_v[sl]
            for c in range(HIDDEN):
                idxc_v[c, sl] = acc
                acc = acc + CB
            return _

        lax.fori_loop(0, n_vec, build, None)

        copies = [
            pltpu.async_copy(flat_hbm.at[idxc_v.at[c]], rows_v.at[c], sem)
            for c in range(HIDDEN)
        ]
        for cp in copies:
            cp.wait()
        pltpu.sync_copy(rows_v, out_hbm.at[:, pl.ds(base, b_per_w)])

    return gather_kernel(flat, pos0)


def _mlp_body(ht_ref, idx_ref, tail_ref, w_ref, b_ref, o_ref):
    ht = ht_ref[...]
    idx = idx_ref[...]
    tail_sel = idx - VMAIN
    onehot = jnp.where(
        lax.broadcasted_iota(jnp.int32, (NTAIL, idx.shape[1]), 0) == tail_sel,
        1.0,
        0.0,
    )
    htail = lax.dot_general(
        tail_ref[...], onehot, (((1,), (0,)), ((), ())),
        preferred_element_type=jnp.float32,
    )
    ht = jnp.where(idx >= VMAIN, htail, ht)
    h = jnp.maximum(ht, 0.0)
    o_ref[...] = (
        lax.dot_general(
            w_ref[...], h, (((1,), (0,)), ((), ())),
            preferred_element_type=jnp.float32,
        )
        + b_ref[...]
    )


def kernel(x, emb, W2, b2):
    b = x.shape[0]
    idx = x.reshape(b).astype(jnp.int32)
    flat = _sc_detile(emb.T)

    idx_c = jnp.minimum(idx, VMAIN - 1)
    g = idx_c // CB
    pos0 = g * BLK + (idx_c - g * CB)
    ht = _sc_gather_t(flat, pos0)

    tail_t = emb[VMAIN:].T
    yt = pl.pallas_call(
        _mlp_body,
        out_shape=jax.ShapeDtypeStruct((OUT, b), jnp.float32),
    )(ht, idx.reshape(1, b), tail_t, W2, b2.reshape(OUT, 1))
    return yt.T
